# Initial kernel scaffold; baseline (speedup 1.0000x reference)
import jax
import jax.numpy as jnp
from jax.experimental import pallas as pl


def kernel(input, weight, bias):
    raise NotImplementedError("write your pallas kernel here")


# speedup vs baseline: 4.8794x; 4.8794x over previous
import functools

import jax
import jax.numpy as jnp
from jax import lax
from jax.experimental import pallas as pl
from jax.experimental.pallas import tpu as pltpu
from jax.experimental.pallas import tpu_sc as plsc

_B, _N, _D, _K = 8, 2048, 128, 16
_RT = 256
_L = 16
_NC, _NS = 2, 16
_NW = _NC * _NS
_RPW = (_B * _N) // _NW
_C = 8
_NCHUNK = _RPW // _C


def _tc_body(xf_ref, xt_ref, w_ref, sup_ref, idx_ref):
    b = pl.program_id(0)
    xf = xf_ref[0]
    xt = xt_ref[0]
    w = w_ref[...]

    sup_ref[0] = jnp.dot(xt, w, preferred_element_type=jnp.float32)

    inner = lax.dot_general(xt, xf, (((1,), (1,)), ((), ())),
                            preferred_element_type=jnp.float32)
    xx_f = jnp.sum(xf * xf, axis=1)
    xx_t = jnp.sum(xt * xt, axis=1)
    dist = -1.0 * (xx_t[:, None] + xx_f[None, :] - 2.0 * inner)

    col = lax.broadcasted_iota(jnp.int32, (_RT, _N), 1)
    kcol = lax.broadcasted_iota(jnp.int32, (_RT, _K), 1)
    neg = jnp.float32(-jnp.inf)

    def step(t, carry):
        d, idx = carry
        m = jnp.max(d, axis=1, keepdims=True)
        fi = jnp.min(jnp.where(d == m, col, _N), axis=1)
        idx = jnp.where(kcol == t, fi[:, None], idx)
        d = jnp.where(col == fi[:, None], neg, d)
        return d, idx

    idx0 = jnp.zeros((_RT, _K), jnp.int32)
    _, idx = lax.fori_loop(0, _K, step, (dist, idx0))
    idx_ref[0] = idx + b * _N


def _tc_call(x, w):
    grid = (_B, _N // _RT)
    return pl.pallas_call(
        _tc_body,
        grid=grid,
        in_specs=[
            pl.BlockSpec((1, _N, _D), lambda b, t: (b, 0, 0)),
            pl.BlockSpec((1, _RT, _D), lambda b, t: (b, t, 0)),
            pl.BlockSpec((_D, _D), lambda b, t: (0, 0)),
        ],
        out_specs=[
            pl.BlockSpec((1, _RT, _D), lambda b, t: (b, t, 0)),
            pl.BlockSpec((1, _RT, _K), lambda b, t: (b, t, 0)),
        ],
        out_shape=[
            jax.ShapeDtypeStruct((_B, _N, _D), jnp.float32),
            jax.ShapeDtypeStruct((_B, _N, _K), jnp.int32),
        ],
    )(x, x, w)


def _sc_body(sup_hbm, gidx_hbm, bias_hbm, out_hbm,
             idx_v, rows_v, out_v, bias_v, sem):
    wid = lax.axis_index("s") * _NC + lax.axis_index("c")
    pltpu.sync_copy(bias_hbm, bias_v)

    def chunk(ci, carry):
        base = wid * _RPW + ci * _C
        pltpu.sync_copy(gidx_hbm.at[pl.ds(base * _K, _C * _K)], idx_v)
        pltpu.async_copy(sup_hbm.at[idx_v], rows_v, sem).wait()
        for r in range(_C):
            for dc in range(_D // _L):
                sl = pl.ds(dc * _L, _L)
                acc = rows_v[r * _K, sl]
                for j in range(1, _K):
                    acc = acc + rows_v[r * _K + j, sl]
                out_v[r, sl] = acc * (1.0 / _K) + bias_v[sl]
        pltpu.sync_copy(out_v, out_hbm.at[pl.ds(base, _C)])
        return carry

    lax.fori_loop(0, _NCHUNK, chunk, 0)


def _sc_call(sup_flat, gidx_flat, bias):
    mesh = plsc.VectorSubcoreMesh(core_axis_name="c", subcore_axis_name="s",
                                  num_cores=_NC, num_subcores=_NS)
    fn = pl.kernel(
        _sc_body,
        out_type=jax.ShapeDtypeStruct((_B * _N, _D), jnp.float32),
        mesh=mesh,
        scratch_types=[
            pltpu.VMEM((_C * _K,), jnp.int32),
            pltpu.VMEM((_C * _K, _D), jnp.float32),
            pltpu.VMEM((_C, _D), jnp.float32),
            pltpu.VMEM((_D,), jnp.float32),
            pltpu.SemaphoreType.DMA,
        ],
    )
    return fn(sup_flat, gidx_flat, bias)


def kernel(input, weight, bias):
    sup, gidx = _tc_call(input, weight)
    out = _sc_call(sup.reshape(_B * _N, _D), gidx.reshape(-1), bias)
    return out.reshape(_B, _N, _D)


# speedup vs baseline: 7.0755x; 1.4501x over previous
import functools

import jax
import jax.numpy as jnp
from jax import lax
from jax.experimental import pallas as pl
from jax.experimental.pallas import tpu as pltpu
from jax.experimental.pallas import tpu_sc as plsc

_B, _N, _D, _K = 8, 2048, 128, 16
_RT = 256
_L = 16
_NC, _NS = 2, 16
_NW = _NC * _NS
_RPW = (_B * _N) // _NW
_C = 8
_NCHUNK = _RPW // _C


def _tc_body(xf_ref, xt_ref, w_ref, sup_ref, idx_ref, xx_ref):
    b = pl.program_id(0)
    t_id = pl.program_id(1)
    xf = xf_ref[0]
    xt = xt_ref[0]
    w = w_ref[...]

    sup_ref[0] = jnp.dot(xt, w, preferred_element_type=jnp.float32)

    @pl.when(t_id == 0)
    def _():
        xx_ref[...] = jnp.sum(xf * xf, axis=1, keepdims=True)

    inner = lax.dot_general(xt, xf, (((1,), (1,)), ((), ())),
                            preferred_element_type=jnp.float32)
    xx_f = xx_ref[...][:, 0]
    xx_t = xx_ref[pl.ds(t_id * _RT, _RT), 0]
    dist = -1.0 * (xx_t[:, None] + xx_f[None, :] - 2.0 * inner)

    col = lax.broadcasted_iota(jnp.int32, (_RT, _N), 1)
    kcol = lax.broadcasted_iota(jnp.int32, (_RT, _K), 1)
    neg = jnp.float32(-jnp.inf)

    d = dist
    m = jnp.max(d, axis=1, keepdims=True)
    fis = []
    for _t in range(_K):
        fi = jnp.min(jnp.where(d == m, col, _N), axis=1)
        fis.append(fi[:, None])
        if _t + 1 < _K:
            d = jnp.where(col == fi[:, None], neg, d)
            m = jnp.max(d, axis=1, keepdims=True)
    idx = jnp.concatenate(fis, axis=1)
    idx_ref[0] = idx + b * _N


def _tc_call(x, w):
    grid = (_B, _N // _RT)
    return pl.pallas_call(
        _tc_body,
        grid=grid,
        in_specs=[
            pl.BlockSpec((1, _N, _D), lambda b, t: (b, 0, 0)),
            pl.BlockSpec((1, _RT, _D), lambda b, t: (b, t, 0)),
            pl.BlockSpec((_D, _D), lambda b, t: (0, 0)),
        ],
        out_specs=[
            pl.BlockSpec((1, _RT, _D), lambda b, t: (b, t, 0)),
            pl.BlockSpec((1, _RT, _K), lambda b, t: (b, t, 0)),
        ],
        out_shape=[
            jax.ShapeDtypeStruct((_B, _N, _D), jnp.float32),
            jax.ShapeDtypeStruct((_B, _N, _K), jnp.int32),
        ],
        scratch_shapes=[pltpu.VMEM((_N, 1), jnp.float32)],
    )(x, x, w)


def _sc_body(sup_hbm, gidx_hbm, bias_hbm, out_hbm,
             idx_v, rows_v, out_v, bias_v, sem):
    wid = lax.axis_index("s") * _NC + lax.axis_index("c")
    pltpu.sync_copy(bias_hbm, bias_v)

    def chunk(ci, carry):
        base = wid * _RPW + ci * _C
        pltpu.sync_copy(gidx_hbm.at[pl.ds(base * _K, _C * _K)], idx_v)
        pltpu.async_copy(sup_hbm.at[idx_v], rows_v, sem).wait()
        for r in range(_C):
            for dc in range(_D // _L):
                sl = pl.ds(dc * _L, _L)
                acc = rows_v[r * _K, sl]
                for j in range(1, _K):
                    acc = acc + rows_v[r * _K + j, sl]
                out_v[r, sl] = acc * (1.0 / _K) + bias_v[sl]
        pltpu.sync_copy(out_v, out_hbm.at[pl.ds(base, _C)])
        return carry

    lax.fori_loop(0, _NCHUNK, chunk, 0)


def _sc_call(sup_flat, gidx_flat, bias):
    mesh = plsc.VectorSubcoreMesh(core_axis_name="c", subcore_axis_name="s",
                                  num_cores=_NC, num_subcores=_NS)
    fn = pl.kernel(
        _sc_body,
        out_type=jax.ShapeDtypeStruct((_B * _N, _D), jnp.float32),
        mesh=mesh,
        scratch_types=[
            pltpu.VMEM((_C * _K,), jnp.int32),
            pltpu.VMEM((_C * _K, _D), jnp.float32),
            pltpu.VMEM((_C, _D), jnp.float32),
            pltpu.VMEM((_D,), jnp.float32),
            pltpu.SemaphoreType.DMA,
        ],
    )
    return fn(sup_flat, gidx_flat, bias)


def kernel(input, weight, bias):
    sup, gidx = _tc_call(input, weight)
    out = _sc_call(sup.reshape(_B * _N, _D), gidx.reshape(-1), bias)
    return out.reshape(_B, _N, _D)


# speedup vs baseline: 8.4721x; 1.1974x over previous
import functools

import jax
import jax.numpy as jnp
from jax import lax
from jax.experimental import pallas as pl
from jax.experimental.pallas import tpu as pltpu
from jax.experimental.pallas import tpu_sc as plsc

_B, _N, _D, _K = 8, 2048, 128, 16
_RT = 256
_L = 16
_NC, _NS = 2, 16
_NW = _NC * _NS
_RPW = (_B * _N) // _NW
_C = 8
_NCHUNK = _RPW // _C


def _tc_body(xf_ref, xt_ref, w_ref, sup_ref, idx_ref, xx_ref):
    b = pl.program_id(0)
    t_id = pl.program_id(1)
    xf = xf_ref[0]
    xt = xt_ref[0]
    w = w_ref[...]

    sup_ref[0] = jnp.dot(xt, w, preferred_element_type=jnp.float32)

    @pl.when(t_id == 0)
    def _():
        xx_ref[...] = jnp.sum(xf * xf, axis=1, keepdims=True)

    inner = lax.dot_general(xt, xf, (((1,), (1,)), ((), ())),
                            preferred_element_type=jnp.float32)
    xx_f = xx_ref[...][:, 0]
    xx_t = xx_ref[pl.ds(t_id * _RT, _RT), 0]
    dist = -1.0 * (xx_t[:, None] + xx_f[None, :] - 2.0 * inner)

    col = lax.broadcasted_iota(jnp.int32, (_RT, _N), 1)
    kcol = lax.broadcasted_iota(jnp.int32, (_RT, _K), 1)
    neg = jnp.float32(-jnp.inf)

    d = dist
    m = jnp.max(d, axis=1, keepdims=True)
    fis = []
    for _t in range(_K):
        fi = jnp.min(jnp.where(d == m, col, _N), axis=1)
        fis.append(fi[:, None])
        if _t + 1 < _K:
            d = jnp.where(col == fi[:, None], neg, d)
            m = jnp.max(d, axis=1, keepdims=True)
    idx = jnp.concatenate(fis, axis=1)
    idx_ref[0] = idx + b * _N


def _tc_call(x, w):
    grid = (_B, _N // _RT)
    return pl.pallas_call(
        _tc_body,
        grid=grid,
        in_specs=[
            pl.BlockSpec((1, _N, _D), lambda b, t: (b, 0, 0)),
            pl.BlockSpec((1, _RT, _D), lambda b, t: (b, t, 0)),
            pl.BlockSpec((_D, _D), lambda b, t: (0, 0)),
        ],
        out_specs=[
            pl.BlockSpec((1, _RT, _D), lambda b, t: (b, t, 0)),
            pl.BlockSpec((1, _RT, _K), lambda b, t: (b, t, 0)),
        ],
        out_shape=[
            jax.ShapeDtypeStruct((_B, _N, _D), jnp.float32),
            jax.ShapeDtypeStruct((_B, _N, _K), jnp.int32),
        ],
        scratch_shapes=[pltpu.VMEM((_N, 1), jnp.float32)],
    )(x, x, w)


_NB = 4


def _sc_body(sup_hbm, gidx_hbm, bias_hbm, out_hbm,
             idx_v, rows0, rows1, rows2, rows3, out_v, bias_v,
             sem0, sem1, sem2, sem3):
    wid = lax.axis_index("s") * _NC + lax.axis_index("c")
    base_row = wid * _RPW
    rows = (rows0, rows1, rows2, rows3)
    sems = (sem0, sem1, sem2, sem3)

    pltpu.sync_copy(bias_hbm, bias_v)
    pltpu.sync_copy(gidx_hbm.at[pl.ds(base_row * _K, _RPW * _K)], idx_v)

    def gather_desc(c, buf, sem):
        return pltpu.make_async_copy(
            sup_hbm.at[idx_v.at[pl.ds(c * (_C * _K), _C * _K)]], buf, sem)

    for p in range(_NB - 1):
        gather_desc(p, rows[p], sems[p]).start()

    def accum(buf):
        def rbody(r, carry):
            ro = r * _K
            for dc in range(_D // _L):
                sl = pl.ds(dc * _L, _L)
                acc = buf[ro, sl]
                for j in range(1, _K):
                    acc = acc + buf[ro + j, sl]
                out_v[r, sl] = acc * (1.0 / _K) + bias_v[sl]
            return carry
        lax.fori_loop(0, _C, rbody, 0)

    def group(g, carry):
        c0 = g * _NB
        for p in range(_NB):
            c = c0 + p
            nxt = c + (_NB - 1)
            nbuf = (p + _NB - 1) % _NB

            @pl.when(nxt < _NCHUNK)
            def _():
                gather_desc(nxt, rows[nbuf], sems[nbuf]).start()

            gather_desc(c, rows[p], sems[p]).wait()
            accum(rows[p])
            pltpu.sync_copy(out_v, out_hbm.at[pl.ds(base_row + c * _C, _C)])
        return carry

    lax.fori_loop(0, _NCHUNK // _NB, group, 0)


def _sc_call(sup_flat, gidx_flat, bias):
    mesh = plsc.VectorSubcoreMesh(core_axis_name="c", subcore_axis_name="s",
                                  num_cores=_NC, num_subcores=_NS)
    fn = pl.kernel(
        _sc_body,
        out_type=jax.ShapeDtypeStruct((_B * _N, _D), jnp.float32),
        mesh=mesh,
        scratch_types=[
            pltpu.VMEM((_RPW * _K,), jnp.int32),
            pltpu.VMEM((_C * _K, _D), jnp.float32),
            pltpu.VMEM((_C * _K, _D), jnp.float32),
            pltpu.VMEM((_C * _K, _D), jnp.float32),
            pltpu.VMEM((_C * _K, _D), jnp.float32),
            pltpu.VMEM((_C, _D), jnp.float32),
            pltpu.VMEM((_D,), jnp.float32),
            pltpu.SemaphoreType.DMA,
            pltpu.SemaphoreType.DMA,
            pltpu.SemaphoreType.DMA,
            pltpu.SemaphoreType.DMA,
        ],
    )
    return fn(sup_flat, gidx_flat, bias)


def kernel(input, weight, bias):
    sup, gidx = _tc_call(input, weight)
    out = _sc_call(sup.reshape(_B * _N, _D), gidx.reshape(-1), bias)
    return out.reshape(_B, _N, _D)


# speedup vs baseline: 9.1775x; 1.0833x over previous
import functools

import jax
import jax.numpy as jnp
from jax import lax
from jax.experimental import pallas as pl
from jax.experimental.pallas import tpu as pltpu
from jax.experimental.pallas import tpu_sc as plsc

_B, _N, _D, _K = 8, 2048, 128, 16
_BH = 2
_RT = 256
_L = 16
_NC, _NS = 2, 16
_NW = _NC * _NS
_RPW = (_BH * _N) // _NW
_C = 8
_NCHUNK = _RPW // _C


def _tc_body(xf_ref, xt_ref, w_ref, sup_ref, idx_ref, xx_ref):
    b = pl.program_id(0)
    t_id = pl.program_id(1)
    xf = xf_ref[0]
    xt = xt_ref[0]
    w = w_ref[...]

    sup_ref[0] = jnp.dot(xt, w, preferred_element_type=jnp.float32)

    @pl.when(t_id == 0)
    def _():
        xx_ref[...] = jnp.sum(xf * xf, axis=1, keepdims=True)

    inner = lax.dot_general(xt, xf, (((1,), (1,)), ((), ())),
                            preferred_element_type=jnp.float32)
    xx_f = xx_ref[...][:, 0]
    xx_t = xx_ref[pl.ds(t_id * _RT, _RT), 0]
    dist = -1.0 * (xx_t[:, None] + xx_f[None, :] - 2.0 * inner)

    col = lax.broadcasted_iota(jnp.int32, (_RT, _N), 1)
    kcol = lax.broadcasted_iota(jnp.int32, (_RT, _K), 1)
    neg = jnp.float32(-jnp.inf)

    d = dist
    m = jnp.max(d, axis=1, keepdims=True)
    fis = []
    for _t in range(_K):
        fi = jnp.min(jnp.where(d == m, col, _N), axis=1)
        fis.append(fi[:, None])
        if _t + 1 < _K:
            d = jnp.where(col == fi[:, None], neg, d)
            m = jnp.max(d, axis=1, keepdims=True)
    idx = jnp.concatenate(fis, axis=1)
    idx_ref[0] = idx + b * _N


def _tc_call(x, w):
    grid = (_BH, _N // _RT)
    return pl.pallas_call(
        _tc_body,
        grid=grid,
        in_specs=[
            pl.BlockSpec((1, _N, _D), lambda b, t: (b, 0, 0)),
            pl.BlockSpec((1, _RT, _D), lambda b, t: (b, t, 0)),
            pl.BlockSpec((_D, _D), lambda b, t: (0, 0)),
        ],
        out_specs=[
            pl.BlockSpec((1, _RT, _D), lambda b, t: (b, t, 0)),
            pl.BlockSpec((1, _RT, _K), lambda b, t: (b, t, 0)),
        ],
        out_shape=[
            jax.ShapeDtypeStruct((_BH, _N, _D), jnp.float32),
            jax.ShapeDtypeStruct((_BH, _N, _K), jnp.int32),
        ],
        scratch_shapes=[pltpu.VMEM((_N, 1), jnp.float32)],
    )(x, x, w)


_NB = 4


def _sc_body(sup_hbm, gidx_hbm, bias_hbm, out_hbm,
             idx_v, rows0, rows1, rows2, rows3, out_v, bias_v,
             sem0, sem1, sem2, sem3):
    wid = lax.axis_index("s") * _NC + lax.axis_index("c")
    base_row = wid * _RPW
    rows = (rows0, rows1, rows2, rows3)
    sems = (sem0, sem1, sem2, sem3)

    pltpu.sync_copy(bias_hbm, bias_v)
    pltpu.sync_copy(gidx_hbm.at[pl.ds(base_row * _K, _RPW * _K)], idx_v)

    def gather_desc(c, buf, sem):
        return pltpu.make_async_copy(
            sup_hbm.at[idx_v.at[pl.ds(c * (_C * _K), _C * _K)]], buf, sem)

    for p in range(_NB - 1):
        gather_desc(p, rows[p], sems[p]).start()

    def accum(buf):
        def rbody(r, carry):
            ro = r * _K
            for dc in range(_D // _L):
                sl = pl.ds(dc * _L, _L)
                acc = buf[ro, sl]
                for j in range(1, _K):
                    acc = acc + buf[ro + j, sl]
                out_v[r, sl] = acc * (1.0 / _K) + bias_v[sl]
            return carry
        lax.fori_loop(0, _C, rbody, 0)

    def group(g, carry):
        c0 = g * _NB
        for p in range(_NB):
            c = c0 + p
            nxt = c + (_NB - 1)
            nbuf = (p + _NB - 1) % _NB

            @pl.when(nxt < _NCHUNK)
            def _():
                gather_desc(nxt, rows[nbuf], sems[nbuf]).start()

            gather_desc(c, rows[p], sems[p]).wait()
            accum(rows[p])
            pltpu.sync_copy(out_v, out_hbm.at[pl.ds(base_row + c * _C, _C)])
        return carry

    lax.fori_loop(0, _NCHUNK // _NB, group, 0)


def _sc_call(sup_flat, gidx_flat, bias):
    mesh = plsc.VectorSubcoreMesh(core_axis_name="c", subcore_axis_name="s",
                                  num_cores=_NC, num_subcores=_NS)
    fn = pl.kernel(
        _sc_body,
        out_type=jax.ShapeDtypeStruct((_BH * _N, _D), jnp.float32),
        mesh=mesh,
        scratch_types=[
            pltpu.VMEM((_RPW * _K,), jnp.int32),
            pltpu.VMEM((_C * _K, _D), jnp.float32),
            pltpu.VMEM((_C * _K, _D), jnp.float32),
            pltpu.VMEM((_C * _K, _D), jnp.float32),
            pltpu.VMEM((_C * _K, _D), jnp.float32),
            pltpu.VMEM((_C, _D), jnp.float32),
            pltpu.VMEM((_D,), jnp.float32),
            pltpu.SemaphoreType.DMA,
            pltpu.SemaphoreType.DMA,
            pltpu.SemaphoreType.DMA,
            pltpu.SemaphoreType.DMA,
        ],
    )
    return fn(sup_flat, gidx_flat, bias)


def kernel(input, weight, bias):
    outs = []
    for h in range(_B // _BH):
        xh = lax.slice_in_dim(input, h * _BH, (h + 1) * _BH, axis=0)
        sup, gidx = _tc_call(xh, weight)
        outs.append(_sc_call(sup.reshape(_BH * _N, _D), gidx.reshape(-1), bias))
    return jnp.concatenate(outs, axis=0).reshape(_B, _N, _D)


# speedup vs baseline: 10.1727x; 1.1084x over previous
import functools

import jax
import jax.numpy as jnp
from jax import lax
from jax.experimental import pallas as pl
from jax.experimental.pallas import tpu as pltpu
from jax.experimental.pallas import tpu_sc as plsc

_B, _N, _D, _K = 8, 2048, 128, 16
_BH = 2
_RT = 512
_L = 16
_NC, _NS = 2, 16
_NW = _NC * _NS
_RPW = (_BH * _N) // _NW
_C = 8
_NCHUNK = _RPW // _C


def _tc_body(xf_ref, xt_ref, w_ref, sup_ref, idx_ref, xx_ref):
    b = pl.program_id(0)
    t_id = pl.program_id(1)
    xf = xf_ref[0]
    xt = xt_ref[0]
    w = w_ref[...]

    sup_ref[0] = jnp.dot(xt, w, preferred_element_type=jnp.float32)

    @pl.when(t_id == 0)
    def _():
        xx_ref[...] = jnp.sum(xf * xf, axis=1, keepdims=True)

    inner = lax.dot_general(xt, xf, (((1,), (1,)), ((), ())),
                            preferred_element_type=jnp.float32)
    xx_f = xx_ref[...][:, 0]
    xx_t = xx_ref[pl.ds(t_id * _RT, _RT), 0]
    dist = -1.0 * (xx_t[:, None] + xx_f[None, :] - 2.0 * inner)

    col = lax.broadcasted_iota(jnp.int32, (_RT, _N), 1)
    kcol = lax.broadcasted_iota(jnp.int32, (_RT, _K), 1)
    neg = jnp.float32(-jnp.inf)

    rowv = lax.broadcasted_iota(jnp.int32, (_RT, 1), 0) + t_id * _RT
    d = jnp.where(col == rowv, neg, dist)
    m = jnp.max(d, axis=1, keepdims=True)
    fis = [rowv]
    for _t in range(1, _K):
        fi = jnp.min(jnp.where(d == m, col, _N), axis=1)
        fis.append(fi[:, None])
        if _t + 1 < _K:
            d = jnp.where(col == fi[:, None], neg, d)
            m = jnp.max(d, axis=1, keepdims=True)
    idx = jnp.concatenate(fis, axis=1)
    idx_ref[0] = idx + b * _N


def _tc_call(x, w):
    grid = (_BH, _N // _RT)
    return pl.pallas_call(
        _tc_body,
        grid=grid,
        in_specs=[
            pl.BlockSpec((1, _N, _D), lambda b, t: (b, 0, 0)),
            pl.BlockSpec((1, _RT, _D), lambda b, t: (b, t, 0)),
            pl.BlockSpec((_D, _D), lambda b, t: (0, 0)),
        ],
        out_specs=[
            pl.BlockSpec((1, _RT, _D), lambda b, t: (b, t, 0)),
            pl.BlockSpec((1, _RT, _K), lambda b, t: (b, t, 0)),
        ],
        out_shape=[
            jax.ShapeDtypeStruct((_BH, _N, _D), jnp.float32),
            jax.ShapeDtypeStruct((_BH, _N, _K), jnp.int32),
        ],
        scratch_shapes=[pltpu.VMEM((_N, 1), jnp.float32)],
    )(x, x, w)


_NB = 4


def _sc_body(sup_hbm, gidx_hbm, bias_hbm, out_hbm,
             idx_v, rows0, rows1, rows2, rows3, out_v, bias_v,
             sem0, sem1, sem2, sem3):
    wid = lax.axis_index("s") * _NC + lax.axis_index("c")
    base_row = wid * _RPW
    rows = (rows0, rows1, rows2, rows3)
    sems = (sem0, sem1, sem2, sem3)

    pltpu.sync_copy(bias_hbm, bias_v)
    pltpu.sync_copy(gidx_hbm.at[pl.ds(base_row * _K, _RPW * _K)], idx_v)

    def gather_desc(c, buf, sem):
        return pltpu.make_async_copy(
            sup_hbm.at[idx_v.at[pl.ds(c * (_C * _K), _C * _K)]], buf, sem)

    for p in range(_NB - 1):
        gather_desc(p, rows[p], sems[p]).start()

    def accum(buf):
        def rbody(r, carry):
            ro = r * _K
            for dc in range(_D // _L):
                sl = pl.ds(dc * _L, _L)
                acc = buf[ro, sl]
                for j in range(1, _K):
                    acc = acc + buf[ro + j, sl]
                out_v[r, sl] = acc * (1.0 / _K) + bias_v[sl]
            return carry
        lax.fori_loop(0, _C, rbody, 0)

    def group(g, carry):
        c0 = g * _NB
        for p in range(_NB):
            c = c0 + p
            nxt = c + (_NB - 1)
            nbuf = (p + _NB - 1) % _NB

            @pl.when(nxt < _NCHUNK)
            def _():
                gather_desc(nxt, rows[nbuf], sems[nbuf]).start()

            gather_desc(c, rows[p], sems[p]).wait()
            accum(rows[p])
            pltpu.sync_copy(out_v, out_hbm.at[pl.ds(base_row + c * _C, _C)])
        return carry

    lax.fori_loop(0, _NCHUNK // _NB, group, 0)


def _sc_call(sup_flat, gidx_flat, bias):
    mesh = plsc.VectorSubcoreMesh(core_axis_name="c", subcore_axis_name="s",
                                  num_cores=_NC, num_subcores=_NS)
    fn = pl.kernel(
        _sc_body,
        out_type=jax.ShapeDtypeStruct((_BH * _N, _D), jnp.float32),
        mesh=mesh,
        scratch_types=[
            pltpu.VMEM((_RPW * _K,), jnp.int32),
            pltpu.VMEM((_C * _K, _D), jnp.float32),
            pltpu.VMEM((_C * _K, _D), jnp.float32),
            pltpu.VMEM((_C * _K, _D), jnp.float32),
            pltpu.VMEM((_C * _K, _D), jnp.float32),
            pltpu.VMEM((_C, _D), jnp.float32),
            pltpu.VMEM((_D,), jnp.float32),
            pltpu.SemaphoreType.DMA,
            pltpu.SemaphoreType.DMA,
            pltpu.SemaphoreType.DMA,
            pltpu.SemaphoreType.DMA,
        ],
    )
    return fn(sup_flat, gidx_flat, bias)


def kernel(input, weight, bias):
    outs = []
    for h in range(_B // _BH):
        xh = lax.slice_in_dim(input, h * _BH, (h + 1) * _BH, axis=0)
        sup, gidx = _tc_call(xh, weight)
        outs.append(_sc_call(sup.reshape(_BH * _N, _D), gidx.reshape(-1), bias))
    return jnp.concatenate(outs, axis=0).reshape(_B, _N, _D)
